# Initial kernel scaffold; baseline (speedup 1.0000x reference)
import jax
import jax.numpy as jnp
from jax.experimental import pallas as pl


def kernel(table, distances):
    raise NotImplementedError("write your pallas kernel here")


# speedup vs baseline: 6.1359x; 6.1359x over previous
import functools

import jax
import jax.numpy as jnp
from jax import lax
from jax.experimental import pallas as pl
from jax.experimental.pallas import tpu as pltpu
from jax.experimental.pallas import tpu_sc as plsc

DIM = 64
WINDOW_SIZE = 128

NC = 2
NS = 16
NW = NC * NS
LANES = 16

CHUNK = 128
GPB = 2
BLK = CHUNK * GPB


def _make_lookup(B):
    assert B % NW == 0
    per_w = B // NW
    assert per_w % BLK == 0
    nblk = per_w // BLK
    nchunk = per_w // CHUNK
    mesh = plsc.VectorSubcoreMesh(core_axis_name="c", subcore_axis_name="s")

    @functools.partial(
        pl.kernel,
        mesh=mesh,
        out_type=jax.ShapeDtypeStruct((B, DIM), jnp.float32),
        scratch_types=[
            pltpu.VMEM_SHARED((257, DIM), jnp.float32),
            pltpu.VMEM((nchunk, CHUNK), jnp.int32),
            pltpu.VMEM((CHUNK, DIM), jnp.float32),
            pltpu.VMEM((CHUNK, DIM), jnp.float32),
            pltpu.SemaphoreType.DMA,
            pltpu.SemaphoreType.DMA,
        ],
    )
    def lookup(table_hbm, dist_hbm, out_hbm, tbl_sh, idx_v, rows_a, rows_b,
               sem_a, sem_b):
        sid = lax.axis_index("s")
        wid = sid * NC + lax.axis_index("c")
        base = wid * per_w

        @pl.when(sid == 0)
        def _():
            pltpu.sync_copy(table_hbm, tbl_sh)

        pltpu.sync_copy(dist_hbm.at[wid], idx_v)
        plsc.subcore_barrier()

        def clip_row(j, carry):
            row = idx_v.at[j]
            for k in range(CHUNK // LANES):
                sl = pl.ds(k * LANES, LANES)
                v = row[sl]
                row[sl] = jnp.clip(v, -WINDOW_SIZE, WINDOW_SIZE) + WINDOW_SIZE
            return carry
        lax.fori_loop(0, nchunk, clip_row, 0)

        def outer(i, carry):
            c0 = 2 * i
            h_a = pltpu.async_copy(tbl_sh.at[idx_v.at[c0]], rows_a, sem_a)
            h_b = pltpu.async_copy(tbl_sh.at[idx_v.at[c0 + 1]], rows_b, sem_b)
            h_a.wait()
            pltpu.sync_copy(
                rows_a, out_hbm.at[pl.ds(base + c0 * CHUNK, CHUNK)])
            h_b.wait()
            pltpu.sync_copy(
                rows_b, out_hbm.at[pl.ds(base + (c0 + 1) * CHUNK, CHUNK)])
            return carry
        lax.fori_loop(0, nchunk // 2, outer, 0)

    return lookup


def kernel(table, distances):
    shape = distances.shape
    B = distances.size
    d = distances.astype(jnp.int32).reshape(NW, B // (NW * CHUNK), CHUNK)
    out = _make_lookup(B)(table.astype(jnp.float32), d)
    return out.reshape(*shape, DIM)
